# baseline (speedup 1.0000x reference)
import functools

import jax
import jax.numpy as jnp
from jax import lax
from jax.experimental import pallas as pl
from jax.experimental.pallas import tpu as pltpu
from jax.experimental.pallas import tpu_sc as plsc

NC = 2
NS = 16
L = 16
NW = NC * NS

D0, D1, D2 = 2, 8192, 2048
N = D0 * D1 * D2
ROWS = D0 * D1
RPW = ROWS // NW
VPR = D2 // L

RPC1 = 16
NCHUNK1 = RPW // RPC1
VPC1 = RPC1 * D2 // L

RPC2 = 8
NCHUNK2 = RPW // RPC2
VPC2 = RPC2 * D2 // L
NG2 = VPC2 // L

_mesh = plsc.VectorSubcoreMesh(core_axis_name="c", subcore_axis_name="s")
_params = pltpu.CompilerParams(needs_layout_passes=False)


def _wid():
    return lax.axis_index("c") * NS + lax.axis_index("s")


def _splat_j(v, j):
    idx = jnp.full((L, 1), j, jnp.int32)
    dn = lax.GatherDimensionNumbers(
        offset_dims=(), collapsed_slice_dims=(0,), start_index_map=(0,)
    )
    return lax.gather(
        v, idx, dn, (1,), mode=lax.GatherScatterMode.PROMISE_IN_BOUNDS
    )


def _splat_last(v):
    return _splat_j(v, L - 1)


@functools.partial(
    pl.kernel,
    out_type=jax.ShapeDtypeStruct((NW, L), jnp.float32),
    mesh=_mesh,
    compiler_params=_params,
    scratch_types=[
        pltpu.VMEM((RPC1, D2), jnp.float32),
        pltpu.VMEM((RPC1, D2), jnp.float32),
        pltpu.VMEM((L,), jnp.float32),
        pltpu.SemaphoreType.DMA,
        pltpu.SemaphoreType.DMA,
    ],
)
def _k1_totals(x_hbm, tot_hbm, buf0, buf1, tvm, is0, is1):
    wid = _wid()
    d = wid // NS
    row0 = (wid % NS) * RPW
    bufs = (buf0, buf1)
    isem = (is0, is1)

    def in_slice(ci):
        return x_hbm.at[d, pl.ds(row0 + ci * RPC1, RPC1), :]

    pltpu.async_copy(in_slice(0), buf0, is0)
    pltpu.async_copy(in_slice(1), buf1, is1)

    def pair_body(k, accs):
        for b in range(2):
            ci = k * 2 + b
            buf = bufs[b]
            pltpu.make_async_copy(in_slice(0), buf, isem[b]).wait()

            @plsc.parallel_loop(0, VPC1 // 4, unroll=2, carry=accs)
            def accs(t, a2):
                i = t >> 5
                c = (t & 31) * (4 * L)
                a0, a1, a2_, a3 = a2
                return (
                    a0 + buf[i, pl.ds(c, L)],
                    a1 + buf[i, pl.ds(c + L, L)],
                    a2_ + buf[i, pl.ds(c + 2 * L, L)],
                    a3 + buf[i, pl.ds(c + 3 * L, L)],
                )

            @pl.when(ci + 2 < NCHUNK1)
            def _():
                pltpu.async_copy(in_slice(ci + 2), buf, isem[b])

        return accs

    z = jnp.zeros((L,), jnp.float32)
    a0, a1, a2, a3 = lax.fori_loop(0, NCHUNK1 // 2, pair_body, (z, z, z, z))
    total = jnp.sum(a0 + a1 + a2 + a3)
    tvm[...] = jnp.full((L,), total, jnp.float32)
    pltpu.sync_copy(tvm, tot_hbm.at[wid])


@functools.partial(
    pl.kernel,
    out_type=jax.ShapeDtypeStruct((D0, D1, D2), jnp.float32),
    mesh=_mesh,
    compiler_params=_params,
    scratch_types=[
        pltpu.VMEM((RPC2, D2), jnp.float32),
        pltpu.VMEM((RPC2, D2), jnp.float32),
        pltpu.VMEM((RPC2, D2), jnp.float32),
        pltpu.VMEM((RPC2, D2), jnp.float32),
        pltpu.VMEM((VPC2,), jnp.float32),
        pltpu.VMEM((VPC2,), jnp.float32),
        pltpu.VMEM((NG2,), jnp.float32),
        pltpu.VMEM((NW, L), jnp.float32),
        pltpu.VMEM((NW,), jnp.float32),
        pltpu.SemaphoreType.DMA,
        pltpu.SemaphoreType.DMA,
        pltpu.SemaphoreType.DMA,
        pltpu.SemaphoreType.DMA,
        pltpu.SemaphoreType.DMA,
        pltpu.SemaphoreType.DMA,
        pltpu.SemaphoreType.DMA,
        pltpu.SemaphoreType.DMA,
    ],
)
def _k2_scan(
    x_hbm, tot_hbm, out_hbm,
    buf0, buf1, buf2, buf3, offs, gts, gos, tvm, excl,
    is0, is1, is2, is3, os0, os1, os2, os3,
):
    wid = _wid()
    d = wid // NS
    row0 = (wid % NS) * RPW
    iot = lax.iota(jnp.int32, 16)
    zeros = jnp.zeros((L,), jnp.int32)
    bufs = (buf0, buf1, buf2, buf3)
    isem = (is0, is1, is2, is3)
    osem = (os0, os1, os2, os3)

    def in_slice(ci):
        return x_hbm.at[d, pl.ds(row0 + ci * RPC2, RPC2), :]

    def out_slice(ci):
        return out_hbm.at[d, pl.ds(row0 + ci * RPC2, RPC2), :]

    pltpu.sync_copy(tot_hbm, tvm)
    t_a = plsc.load_gather(tvm, [iot, zeros])
    t_b = plsc.load_gather(tvm, [iot + L, zeros])
    s_a = plsc.cumsum(t_a)
    s_b = plsc.cumsum(t_b) + _splat_last(s_a)
    excl[pl.ds(0, L)] = s_a - t_a
    excl[pl.ds(L, L)] = s_b - t_b
    carry0 = plsc.load_gather(excl, [jnp.full((L,), wid, jnp.int32)])

    pltpu.async_copy(in_slice(0), buf0, is0)
    pltpu.async_copy(in_slice(1), buf1, is1)

    def quad_body(k, carry):
        for b in range(4):
            ci = k * 4 + b
            pre = (b + 2) % 4
            buf = bufs[b]

            @pl.when(ci >= 2)
            def _():
                pltpu.make_async_copy(bufs[pre], out_slice(0), osem[pre]).wait()

            @pl.when(ci + 2 < NCHUNK2)
            def _():
                pltpu.async_copy(in_slice(ci + 2), bufs[pre], isem[pre])

            pltpu.make_async_copy(in_slice(0), buf, isem[b]).wait()

            @plsc.parallel_loop(0, NG2, unroll=4)
            def p1(g):
                i = g >> 3
                cb = (g & 7) * (L * L)
                for j in range(L):
                    c = cb + j * L
                    buf[i, pl.ds(c, L)] = plsc.cumsum(buf[i, pl.ds(c, L)])
                tg = plsc.load_gather(
                    buf, [jnp.full((L,), i, jnp.int32), cb + iot * L + (L - 1)]
                )
                sg = plsc.cumsum(tg)
                offs[pl.ds(g * L, L)] = sg - tg
                gts[pl.ds(g * L, L)] = _splat_last(sg)

            for gg in range(NG2 // L):
                gt = plsc.load_gather(gts, [(gg * L + iot) * L])
                sgt = plsc.cumsum(gt)
                gos[pl.ds(gg * L, L)] = sgt - gt + carry
                carry = carry + _splat_last(sgt)

            @plsc.parallel_loop(0, NG2, unroll=4)
            def p3(g):
                i = g >> 3
                cb = (g & 7) * (L * L)
                go = plsc.load_gather(gos, [jnp.full((L,), g, jnp.int32)])
                ov = offs[pl.ds(g * L, L)] + go
                for j in range(L):
                    c = cb + j * L
                    buf[i, pl.ds(c, L)] = buf[i, pl.ds(c, L)] + _splat_j(ov, j)

            pltpu.async_copy(buf, out_slice(ci), osem[b])

        return carry

    lax.fori_loop(0, NCHUNK2 // 4, quad_body, carry0)
    pltpu.make_async_copy(buf2, out_slice(0), os2).wait()
    pltpu.make_async_copy(buf3, out_slice(0), os3).wait()


def kernel(x):
    totals = _k1_totals(x)
    return _k2_scan(x, totals)


# speedup vs baseline: 1.2407x; 1.2407x over previous
import functools

import jax
import jax.numpy as jnp
from jax import lax
from jax.experimental import pallas as pl
from jax.experimental.pallas import tpu as pltpu
from jax.experimental.pallas import tpu_sc as plsc

NC = 2
NS = 16
L = 16
NW = NC * NS

D0, D1, D2 = 2, 8192, 2048
N = D0 * D1 * D2
ROWS = D0 * D1
RPW = ROWS // NW
VPR = D2 // L

RPC1 = 16
NCHUNK1 = RPW // RPC1
VPC1 = RPC1 * D2 // L

RPC2 = 8
NCHUNK2 = RPW // RPC2
VPC2 = RPC2 * D2 // L
NG2 = VPC2 // L

_mesh = plsc.VectorSubcoreMesh(core_axis_name="c", subcore_axis_name="s")
_params = pltpu.CompilerParams(needs_layout_passes=False)


def _wid():
    return lax.axis_index("c") * NS + lax.axis_index("s")


def _splat_j(v, j):
    idx = jnp.full((L, 1), j, jnp.int32)
    dn = lax.GatherDimensionNumbers(
        offset_dims=(), collapsed_slice_dims=(0,), start_index_map=(0,)
    )
    return lax.gather(
        v, idx, dn, (1,), mode=lax.GatherScatterMode.PROMISE_IN_BOUNDS
    )


def _splat_last(v):
    return _splat_j(v, L - 1)


@functools.partial(
    pl.kernel,
    out_type=jax.ShapeDtypeStruct((NW, L), jnp.float32),
    mesh=_mesh,
    compiler_params=_params,
    scratch_types=[
        pltpu.VMEM((RPC1, D2), jnp.float32),
        pltpu.VMEM((RPC1, D2), jnp.float32),
        pltpu.VMEM((L,), jnp.float32),
        pltpu.SemaphoreType.DMA,
        pltpu.SemaphoreType.DMA,
    ],
)
def _k1_totals(x_hbm, tot_hbm, buf0, buf1, tvm, is0, is1):
    wid = _wid()
    d = wid // NS
    row0 = (wid % NS) * RPW
    bufs = (buf0, buf1)
    isem = (is0, is1)

    def in_slice(ci):
        return x_hbm.at[d, pl.ds(row0 + ci * RPC1, RPC1), :]

    pltpu.async_copy(in_slice(0), buf0, is0)
    pltpu.async_copy(in_slice(1), buf1, is1)

    def pair_body(k, accs):
        for b in range(2):
            ci = k * 2 + b
            buf = bufs[b]
            pltpu.make_async_copy(in_slice(0), buf, isem[b]).wait()

            @plsc.parallel_loop(0, VPC1 // 4, unroll=2, carry=accs)
            def accs(t, a2):
                i = t >> 5
                c = (t & 31) * (4 * L)
                a0, a1, a2_, a3 = a2
                return (
                    a0 + buf[i, pl.ds(c, L)],
                    a1 + buf[i, pl.ds(c + L, L)],
                    a2_ + buf[i, pl.ds(c + 2 * L, L)],
                    a3 + buf[i, pl.ds(c + 3 * L, L)],
                )

            @pl.when(ci + 2 < NCHUNK1)
            def _():
                pltpu.async_copy(in_slice(ci + 2), buf, isem[b])

        return accs

    z = jnp.zeros((L,), jnp.float32)
    a0, a1, a2, a3 = lax.fori_loop(0, NCHUNK1 // 2, pair_body, (z, z, z, z))
    total = jnp.sum(a0 + a1 + a2 + a3)
    tvm[...] = jnp.full((L,), total, jnp.float32)
    pltpu.sync_copy(tvm, tot_hbm.at[wid])


@functools.partial(
    pl.kernel,
    out_type=jax.ShapeDtypeStruct((D0, D1, D2), jnp.float32),
    mesh=_mesh,
    compiler_params=_params,
    scratch_types=[
        pltpu.VMEM((RPC2, D2), jnp.float32),
        pltpu.VMEM((RPC2, D2), jnp.float32),
        pltpu.VMEM((RPC2, D2), jnp.float32),
        pltpu.VMEM((RPC2, D2), jnp.float32),
        pltpu.VMEM((VPC2,), jnp.float32),
        pltpu.VMEM((VPC2,), jnp.float32),
        pltpu.VMEM((NG2,), jnp.float32),
        pltpu.VMEM((NW, L), jnp.float32),
        pltpu.VMEM((NW,), jnp.float32),
        pltpu.SemaphoreType.DMA,
        pltpu.SemaphoreType.DMA,
        pltpu.SemaphoreType.DMA,
        pltpu.SemaphoreType.DMA,
        pltpu.SemaphoreType.DMA,
        pltpu.SemaphoreType.DMA,
        pltpu.SemaphoreType.DMA,
        pltpu.SemaphoreType.DMA,
    ],
)
def _k2_scan(
    x_hbm, tot_hbm, out_hbm,
    buf0, buf1, buf2, buf3, offs, gts, gos, tvm, excl,
    is0, is1, is2, is3, os0, os1, os2, os3,
):
    wid = _wid()
    d = wid // NS
    row0 = (wid % NS) * RPW
    iot = lax.iota(jnp.int32, 16)
    zeros = jnp.zeros((L,), jnp.int32)
    bufs = (buf0, buf1, buf2, buf3)
    isem = (is0, is1, is2, is3)
    osem = (os0, os1, os2, os3)

    def in_slice(ci):
        return x_hbm.at[d, pl.ds(row0 + ci * RPC2, RPC2), :]

    def out_slice(ci):
        return out_hbm.at[d, pl.ds(row0 + ci * RPC2, RPC2), :]

    pltpu.sync_copy(tot_hbm, tvm)
    t_a = plsc.load_gather(tvm, [iot, zeros])
    t_b = plsc.load_gather(tvm, [iot + L, zeros])
    s_a = plsc.cumsum(t_a)
    s_b = plsc.cumsum(t_b) + _splat_last(s_a)
    excl[pl.ds(0, L)] = s_a - t_a
    excl[pl.ds(L, L)] = s_b - t_b
    carry0 = plsc.load_gather(excl, [jnp.full((L,), wid, jnp.int32)])

    pltpu.async_copy(in_slice(0), buf0, is0)
    pltpu.async_copy(in_slice(1), buf1, is1)

    def quad_body(k, carry):
        for b in range(4):
            ci = k * 4 + b
            pre = (b + 2) % 4
            buf = bufs[b]

            @pl.when(ci >= 2)
            def _():
                pltpu.make_async_copy(bufs[pre], out_slice(0), osem[pre]).wait()

            @pl.when(ci + 2 < NCHUNK2)
            def _():
                pltpu.async_copy(in_slice(ci + 2), bufs[pre], isem[pre])

            pltpu.make_async_copy(in_slice(0), buf, isem[b]).wait()

            @plsc.parallel_loop(0, NG2, unroll=2)
            def p1(g):
                i = g >> 3
                cb = (g & 7) * (L * L)
                for j in range(L):
                    c = cb + j * L
                    buf[i, pl.ds(c, L)] = plsc.cumsum(buf[i, pl.ds(c, L)])

            @plsc.parallel_loop(0, NG2, unroll=2)
            def p1b(g):
                i = g >> 3
                cb = (g & 7) * (L * L)
                tg = plsc.load_gather(
                    buf, [jnp.full((L,), i, jnp.int32), cb + iot * L + (L - 1)]
                )
                sg = plsc.cumsum(tg)
                offs[pl.ds(g * L, L)] = sg - tg
                gts[pl.ds(g * L, L)] = _splat_last(sg)

            for gg in range(NG2 // L):
                gt = plsc.load_gather(gts, [(gg * L + iot) * L])
                sgt = plsc.cumsum(gt)
                gos[pl.ds(gg * L, L)] = sgt - gt + carry
                carry = carry + _splat_last(sgt)

            @plsc.parallel_loop(0, NG2, unroll=2)
            def p3(g):
                i = g >> 3
                cb = (g & 7) * (L * L)
                go = plsc.load_gather(gos, [jnp.full((L,), g, jnp.int32)])
                ov = offs[pl.ds(g * L, L)] + go
                for j in range(L):
                    c = cb + j * L
                    buf[i, pl.ds(c, L)] = buf[i, pl.ds(c, L)] + _splat_j(ov, j)

            pltpu.async_copy(buf, out_slice(ci), osem[b])

        return carry

    lax.fori_loop(0, NCHUNK2 // 4, quad_body, carry0)
    pltpu.make_async_copy(buf2, out_slice(0), os2).wait()
    pltpu.make_async_copy(buf3, out_slice(0), os3).wait()


def kernel(x):
    totals = _k1_totals(x)
    return _k2_scan(x, totals)


# speedup vs baseline: 1.3035x; 1.0507x over previous
import functools

import jax
import jax.numpy as jnp
from jax import lax
from jax.experimental import pallas as pl
from jax.experimental.pallas import tpu as pltpu
from jax.experimental.pallas import tpu_sc as plsc

NC = 2
NS = 16
L = 16
NW = NC * NS

D0, D1, D2 = 2, 8192, 2048
N = D0 * D1 * D2
ROWS = D0 * D1
RPW = ROWS // NW
VPR = D2 // L

RPC1 = 16
NCHUNK1 = RPW // RPC1
VPC1 = RPC1 * D2 // L

RPC2 = 8
NCHUNK2 = RPW // RPC2
VPC2 = RPC2 * D2 // L
NG2 = VPC2 // L

_mesh = plsc.VectorSubcoreMesh(core_axis_name="c", subcore_axis_name="s")
_params = pltpu.CompilerParams(needs_layout_passes=False)


def _wid():
    return lax.axis_index("c") * NS + lax.axis_index("s")


def _splat_j(v, j):
    idx = jnp.full((L, 1), j, jnp.int32)
    dn = lax.GatherDimensionNumbers(
        offset_dims=(), collapsed_slice_dims=(0,), start_index_map=(0,)
    )
    return lax.gather(
        v, idx, dn, (1,), mode=lax.GatherScatterMode.PROMISE_IN_BOUNDS
    )


def _splat_last(v):
    return _splat_j(v, L - 1)


@functools.partial(
    pl.kernel,
    out_type=jax.ShapeDtypeStruct((NW, L), jnp.float32),
    mesh=_mesh,
    compiler_params=_params,
    scratch_types=[
        pltpu.VMEM((RPC1, D2), jnp.float32),
        pltpu.VMEM((RPC1, D2), jnp.float32),
        pltpu.VMEM((L,), jnp.float32),
        pltpu.SemaphoreType.DMA,
        pltpu.SemaphoreType.DMA,
    ],
)
def _k1_totals(x_hbm, tot_hbm, buf0, buf1, tvm, is0, is1):
    wid = _wid()
    d = wid // NS
    row0 = (wid % NS) * RPW
    bufs = (buf0, buf1)
    isem = (is0, is1)

    def in_slice(ci):
        return x_hbm.at[d, pl.ds(row0 + ci * RPC1, RPC1), :]

    pltpu.async_copy(in_slice(0), buf0, is0)
    pltpu.async_copy(in_slice(1), buf1, is1)

    def pair_body(k, accs):
        for b in range(2):
            ci = k * 2 + b
            buf = bufs[b]
            pltpu.make_async_copy(in_slice(0), buf, isem[b]).wait()

            @plsc.parallel_loop(0, VPC1 // 4, unroll=2, carry=accs)
            def accs(t, a2):
                i = t >> 5
                c = (t & 31) * (4 * L)
                a0, a1, a2_, a3 = a2
                return (
                    a0 + buf[i, pl.ds(c, L)],
                    a1 + buf[i, pl.ds(c + L, L)],
                    a2_ + buf[i, pl.ds(c + 2 * L, L)],
                    a3 + buf[i, pl.ds(c + 3 * L, L)],
                )

            @pl.when(ci + 2 < NCHUNK1)
            def _():
                pltpu.async_copy(in_slice(ci + 2), buf, isem[b])

        return accs

    z = jnp.zeros((L,), jnp.float32)
    a0, a1, a2, a3 = lax.fori_loop(0, NCHUNK1 // 2, pair_body, (z, z, z, z))
    total = jnp.sum(a0 + a1 + a2 + a3)
    tvm[...] = jnp.full((L,), total, jnp.float32)
    pltpu.sync_copy(tvm, tot_hbm.at[wid])


@functools.partial(
    pl.kernel,
    out_type=jax.ShapeDtypeStruct((D0, D1, D2), jnp.float32),
    mesh=_mesh,
    compiler_params=_params,
    scratch_types=[
        pltpu.VMEM((RPC2, D2), jnp.float32),
        pltpu.VMEM((RPC2, D2), jnp.float32),
        pltpu.VMEM((RPC2, D2), jnp.float32),
        pltpu.VMEM((RPC2, D2), jnp.float32),
        pltpu.VMEM((VPC2,), jnp.float32),
        pltpu.VMEM((VPC2,), jnp.float32),
        pltpu.VMEM((NG2,), jnp.float32),
        pltpu.VMEM((NW, L), jnp.float32),
        pltpu.VMEM((NW,), jnp.float32),
        pltpu.SemaphoreType.DMA,
        pltpu.SemaphoreType.DMA,
        pltpu.SemaphoreType.DMA,
        pltpu.SemaphoreType.DMA,
        pltpu.SemaphoreType.DMA,
        pltpu.SemaphoreType.DMA,
        pltpu.SemaphoreType.DMA,
        pltpu.SemaphoreType.DMA,
    ],
)
def _k2_scan(
    x_hbm, tot_hbm, out_hbm,
    buf0, buf1, buf2, buf3, offs, gts, gos, tvm, excl,
    is0, is1, is2, is3, os0, os1, os2, os3,
):
    wid = _wid()
    d = wid // NS
    row0 = (wid % NS) * RPW
    iot = lax.iota(jnp.int32, 16)
    zeros = jnp.zeros((L,), jnp.int32)
    bufs = (buf0, buf1, buf2, buf3)
    isem = (is0, is1, is2, is3)
    osem = (os0, os1, os2, os3)

    def in_slice(ci):
        return x_hbm.at[d, pl.ds(row0 + ci * RPC2, RPC2), :]

    def out_slice(ci):
        return out_hbm.at[d, pl.ds(row0 + ci * RPC2, RPC2), :]

    pltpu.sync_copy(tot_hbm, tvm)
    t_a = plsc.load_gather(tvm, [iot, zeros])
    t_b = plsc.load_gather(tvm, [iot + L, zeros])
    s_a = plsc.cumsum(t_a)
    s_b = plsc.cumsum(t_b) + _splat_last(s_a)
    excl[pl.ds(0, L)] = s_a - t_a
    excl[pl.ds(L, L)] = s_b - t_b
    carry0 = plsc.load_gather(excl, [jnp.full((L,), wid, jnp.int32)])

    pltpu.async_copy(in_slice(0), buf0, is0)
    pltpu.async_copy(in_slice(1), buf1, is1)

    def quad_body(k, carry):
        for b in range(4):
            ci = k * 4 + b
            pre = (b + 2) % 4
            buf = bufs[b]

            @pl.when(ci >= 2)
            def _():
                pltpu.make_async_copy(bufs[pre], out_slice(0), osem[pre]).wait()

            @pl.when(ci + 2 < NCHUNK2)
            def _():
                pltpu.async_copy(in_slice(ci + 2), bufs[pre], isem[pre])

            pltpu.make_async_copy(in_slice(0), buf, isem[b]).wait()

            @plsc.parallel_loop(0, NG2, unroll=4)
            def p1(g):
                i = g >> 3
                cb = (g & 7) * (L * L)
                for j in range(L):
                    c = cb + j * L
                    buf[i, pl.ds(c, L)] = plsc.cumsum(buf[i, pl.ds(c, L)])

            @plsc.parallel_loop(0, NG2, unroll=4)
            def p1b(g):
                i = g >> 3
                cb = (g & 7) * (L * L)
                tg = plsc.load_gather(
                    buf, [jnp.full((L,), i, jnp.int32), cb + iot * L + (L - 1)]
                )
                sg = plsc.cumsum(tg)
                offs[pl.ds(g * L, L)] = sg - tg
                gts[pl.ds(g * L, L)] = _splat_last(sg)

            for gg in range(NG2 // L):
                gt = plsc.load_gather(gts, [(gg * L + iot) * L])
                sgt = plsc.cumsum(gt)
                gos[pl.ds(gg * L, L)] = sgt - gt + carry
                carry = carry + _splat_last(sgt)

            @plsc.parallel_loop(0, NG2, unroll=2)
            def p3(g):
                i = g >> 3
                cb = (g & 7) * (L * L)
                go = plsc.load_gather(gos, [jnp.full((L,), g, jnp.int32)])
                ov = offs[pl.ds(g * L, L)] + go
                for j in range(L):
                    c = cb + j * L
                    buf[i, pl.ds(c, L)] = buf[i, pl.ds(c, L)] + _splat_j(ov, j)

            pltpu.async_copy(buf, out_slice(ci), osem[b])

        return carry

    lax.fori_loop(0, NCHUNK2 // 4, quad_body, carry0)
    pltpu.make_async_copy(buf2, out_slice(0), os2).wait()
    pltpu.make_async_copy(buf3, out_slice(0), os3).wait()


def kernel(x):
    totals = _k1_totals(x)
    return _k2_scan(x, totals)
